# Initial kernel scaffold; baseline (speedup 1.0000x reference)
import jax
import jax.numpy as jnp
from jax.experimental import pallas as pl


def kernel(logits, targets_full, sup_mask, ignore_mask, kv_indices, kv_num_blocks, pos):
    raise NotImplementedError("write your pallas kernel here")


# speedup vs baseline: 39.8852x; 39.8852x over previous
import jax
import jax.numpy as jnp
from jax.experimental import pallas as pl
from jax.experimental.pallas import tpu as pltpu

_INTERPRET = False


def kernel(logits, targets_full, sup_mask, ignore_mask, kv_indices, kv_num_blocks, pos):
    B, N = sup_mask.shape
    NB, MAXKV = kv_indices.shape[1], kv_indices.shape[2]
    BS = N // NB
    x = logits[..., 0].reshape(1, B * N)
    t = targets_full[..., 0].reshape(1, B * N)
    sup = sup_mask.astype(jnp.float32).reshape(1, B * N)
    ign = ignore_mask.astype(jnp.float32).reshape(1, B * N)
    px = pos[..., 0].reshape(1, B * N)
    py = pos[..., 1].reshape(1, B * N)

    def _body(x_ref, t_ref, sup_ref, ign_ref, px_ref, py_ref, kvi_ref, kvn_ref,
              out_ref, acc_ref):
        b = pl.program_id(0)
        qb = pl.program_id(1)

        @pl.when(jnp.logical_and(b == 0, qb == 0))
        def _init():
            for i in range(8):
                acc_ref[i] = 0.0

        base = b * N + qb * BS
        xq = x_ref[:, pl.ds(base, BS)]
        tq = t_ref[:, pl.ds(base, BS)]
        supq = sup_ref[:, pl.ds(base, BS)]
        ignq = ign_ref[:, pl.ds(base, BS)]
        qx = px_ref[:, pl.ds(base, BS)]
        qy = py_ref[:, pl.ds(base, BS)]

        bce = jnp.maximum(xq, 0.0) - xq * tq + jnp.log1p(jnp.exp(-jnp.abs(xq)))
        acc_ref[0] += jnp.sum(bce * supq)
        acc_ref[1] += jnp.sum(supq)

        qx_c = jnp.broadcast_to(qx, (BS, BS)).T[:, 0:1]
        qy_c = jnp.broadcast_to(qy, (BS, BS)).T[:, 0:1]
        qp_c = jax.nn.sigmoid(jnp.broadcast_to(xq, (BS, BS)).T[:, 0:1])
        unc_c = jnp.broadcast_to((1.0 - supq) * (1.0 - ignq), (BS, BS)).T[:, 0:1]

        rowi = jax.lax.broadcasted_iota(jnp.int32, (BS, BS), 0)
        colj = jax.lax.broadcasted_iota(jnp.int32, (BS, BS), 1)
        diag = rowi == colj

        kvn = kvn_ref[b, qb]
        wsum = jnp.zeros((BS, 1), jnp.float32)
        wp = jnp.zeros((BS, 1), jnp.float32)
        for s in range(MAXKV):
            kb = kvi_ref[b, qb, s]
            kbase = b * N + kb * BS
            kx = px_ref[:, pl.ds(kbase, BS)]
            ky = py_ref[:, pl.ds(kbase, BS)]
            kxl = x_ref[:, pl.ds(kbase, BS)]
            kign = ign_ref[:, pl.ds(kbase, BS)]
            slot_ok = (s < kvn).astype(jnp.float32)
            kvalid = slot_ok * (1.0 - kign)
            dx = qx_c - kx
            dy = qy_c - ky
            d = jnp.sqrt(dx * dx + dy * dy + 1e-12)
            w = jnp.exp(-d)
            w = jnp.where(jnp.logical_and(diag, kb == qb), 0.0, w)
            w = w * kvalid
            wsum += jnp.sum(w, axis=1, keepdims=True)
            wp += jnp.sum(w * jax.nn.sigmoid(kxl), axis=1, keepdims=True)

        kmean = wp / (wsum + 1e-8)
        acc_ref[2 + b] += jnp.sum(((qp_c - kmean) ** 2) * unc_c)
        acc_ref[4 + b] += jnp.sum(unc_c)

        @pl.when(jnp.logical_and(b == B - 1, qb == NB - 1))
        def _final():
            loss_sup = acc_ref[0] / jnp.maximum(acc_ref[1], 1.0)
            g = 0.0
            for bb in range(B):
                g += acc_ref[2 + bb] / jnp.maximum(acc_ref[4 + bb], 1.0)
            out_ref[0] = loss_sup + 10.0 * g / B

    full = pl.BlockSpec((1, B * N), lambda b, q: (0, 0))
    smem = pl.BlockSpec(memory_space=pltpu.SMEM)
    out = pl.pallas_call(
        _body,
        grid=(B, NB),
        in_specs=[full, full, full, full, full, full, smem, smem],
        out_specs=pl.BlockSpec(memory_space=pltpu.SMEM),
        out_shape=jax.ShapeDtypeStruct((1,), jnp.float32),
        scratch_shapes=[pltpu.SMEM((8,), jnp.float32)],
        interpret=_INTERPRET,
    )(x, t, sup, ign, px, py, kv_indices, kv_num_blocks)
    return out[0]
